# baseline (speedup 1.0000x reference)
import functools

import jax
import jax.numpy as jnp
from jax import lax
from jax.experimental import pallas as pl
from jax.experimental.pallas import tpu as pltpu
from jax.experimental.pallas import tpu_sc as plsc

F32 = jnp.float32

NC = 2
NS = 16
NW = NC * NS
C = 80
RB = 40


def _node_proj(x, Ws, Wr, bn=1000):
    N, D = x.shape
    H = Ws.shape[1]

    def body(x_ref, ws_ref, wr_ref, out_ref):
        xb = x_ref[...]
        out_ref[...] = jnp.concatenate(
            [jnp.dot(xb, ws_ref[...], preferred_element_type=F32),
             jnp.dot(xb, wr_ref[...], preferred_element_type=F32)], axis=1)

    return pl.pallas_call(
        body,
        grid=(N // bn,),
        in_specs=[
            pl.BlockSpec((bn, D), lambda i: (i, 0)),
            pl.BlockSpec((D, H), lambda i: (0, 0)),
            pl.BlockSpec((D, H), lambda i: (0, 0)),
        ],
        out_specs=pl.BlockSpec((bn, 2 * H), lambda i: (i, 0)),
        out_shape=jax.ShapeDtypeStruct((N, 2 * H), F32),
    )(x, Ws, Wr)


def _edge_proj(ea2, Wea2, u, Wu, eb_b, be=2000):
    E2, De2 = ea2.shape
    H = Wu.shape[1]

    def body(ea_ref, wea_ref, u_ref, wu_ref, b_ref, out_ref):
        c1 = jnp.dot(u_ref[...], wu_ref[...], preferred_element_type=F32) + b_ref[...]
        c2 = jnp.concatenate([c1, c1], axis=1)
        out_ref[...] = (
            jnp.dot(ea_ref[...], wea_ref[...], preferred_element_type=F32) + c2
        )

    return pl.pallas_call(
        body,
        grid=(E2 // be,),
        in_specs=[
            pl.BlockSpec((be, De2), lambda i: (i, 0)),
            pl.BlockSpec((De2, 2 * H), lambda i: (0, 0)),
            pl.BlockSpec((1, H), lambda i: (0, 0)),
            pl.BlockSpec((H, H), lambda i: (0, 0)),
            pl.BlockSpec((1, H), lambda i: (0, 0)),
        ],
        out_specs=pl.BlockSpec((be, 2 * H), lambda i: (i, 0)),
        out_shape=jax.ShapeDtypeStruct((E2, 2 * H), F32),
    )(ea2, Wea2, u, Wu, eb_b)


def _node_block(aggP, x, Wagg, Wx, u, Wnu, nb_b, W1s, W1r, bn=1000):
    N, D = x.shape
    H = Wagg.shape[1]

    def body(aggP_ref, x_ref, wagg_ref, wx_ref, u_ref, wnu_ref, b_ref,
             w1s_ref, w1r_ref, nsnr_ref, nsum_ref, esum_ref):
        agg = aggP_ref[0] + aggP_ref[1]
        cn = jnp.dot(u_ref[...], wnu_ref[...], preferred_element_type=F32) + b_ref[...]
        n1 = jnp.maximum(
            jnp.dot(agg, wagg_ref[...], preferred_element_type=F32)
            + jnp.dot(x_ref[...], wx_ref[...], preferred_element_type=F32)
            + cn, 0.0)
        nsnr_ref[...] = jnp.concatenate(
            [jnp.dot(n1, w1s_ref[...], preferred_element_type=F32),
             jnp.dot(n1, w1r_ref[...], preferred_element_type=F32)], axis=1)

        @pl.when(pl.program_id(0) == 0)
        def _():
            nsum_ref[...] = jnp.zeros_like(nsum_ref)
            esum_ref[...] = jnp.zeros_like(esum_ref)

        nsum_ref[...] += jnp.sum(n1, axis=0, keepdims=True)
        esum_ref[...] += jnp.sum(agg, axis=0, keepdims=True)

    return pl.pallas_call(
        body,
        grid=(N // bn,),
        in_specs=[
            pl.BlockSpec((NC, bn, H), lambda i: (0, i, 0)),
            pl.BlockSpec((bn, D), lambda i: (i, 0)),
            pl.BlockSpec((H, H), lambda i: (0, 0)),
            pl.BlockSpec((D, H), lambda i: (0, 0)),
            pl.BlockSpec((1, H), lambda i: (0, 0)),
            pl.BlockSpec((H, H), lambda i: (0, 0)),
            pl.BlockSpec((1, H), lambda i: (0, 0)),
            pl.BlockSpec((H, H), lambda i: (0, 0)),
            pl.BlockSpec((H, H), lambda i: (0, 0)),
        ],
        out_specs=[
            pl.BlockSpec((bn, 2 * H), lambda i: (i, 0)),
            pl.BlockSpec((1, H), lambda i: (0, 0)),
            pl.BlockSpec((1, H), lambda i: (0, 0)),
        ],
        out_shape=[
            jax.ShapeDtypeStruct((N, 2 * H), F32),
            jax.ShapeDtypeStruct((1, H), F32),
            jax.ShapeDtypeStruct((1, H), F32),
        ],
    )(aggP, x, Wagg, Wx, u, Wnu, nb_b, W1s, W1r)


def _decoder(e1p, d2, esum, nsum, u, gbW_e, gbW_n, gbW_u, gb_b, W1e2, W1g,
             dec_b1, W22, b22, n_edges, n_nodes, be=2000):
    E2, H2 = e1p.shape
    H = H2 // 2
    O2 = W22.shape[1]

    def body(e1_ref, d_ref, esum_ref, nsum_ref, u_ref, gbe_ref, gbn_ref,
             gbu_ref, gbb_ref, w1e_ref, w1g_ref, b1_ref, w2_ref, b2_ref,
             out_ref):
        g1 = jnp.maximum(
            jnp.dot(esum_ref[...] * (1.0 / n_edges), gbe_ref[...],
                    preferred_element_type=F32)
            + jnp.dot(nsum_ref[...] * (1.0 / n_nodes), gbn_ref[...],
                      preferred_element_type=F32)
            + jnp.dot(u_ref[...], gbu_ref[...], preferred_element_type=F32)
            + gbb_ref[...], 0.0)
        cdec = jnp.dot(g1, w1g_ref[...], preferred_element_type=F32) + b1_ref[...]
        cdec2 = jnp.concatenate([cdec, cdec], axis=1)
        h = jnp.maximum(
            jnp.dot(e1_ref[...], w1e_ref[...], preferred_element_type=F32)
            + d_ref[...] + cdec2, 0.0)
        out_ref[...] = jnp.dot(h, w2_ref[...], preferred_element_type=F32) + b2_ref[...]

    return pl.pallas_call(
        body,
        grid=(E2 // be,),
        in_specs=[
            pl.BlockSpec((be, H2), lambda i: (i, 0)),
            pl.BlockSpec((be, H2), lambda i: (i, 0)),
            pl.BlockSpec((1, H), lambda i: (0, 0)),
            pl.BlockSpec((1, H), lambda i: (0, 0)),
            pl.BlockSpec((1, H), lambda i: (0, 0)),
            pl.BlockSpec((H, H), lambda i: (0, 0)),
            pl.BlockSpec((H, H), lambda i: (0, 0)),
            pl.BlockSpec((H, H), lambda i: (0, 0)),
            pl.BlockSpec((1, H), lambda i: (0, 0)),
            pl.BlockSpec((H2, H2), lambda i: (0, 0)),
            pl.BlockSpec((H, H), lambda i: (0, 0)),
            pl.BlockSpec((1, H), lambda i: (0, 0)),
            pl.BlockSpec((H2, O2), lambda i: (0, 0)),
            pl.BlockSpec((1, O2), lambda i: (0, 0)),
        ],
        out_specs=pl.BlockSpec((be, O2), lambda i: (i, 0)),
        out_shape=jax.ShapeDtypeStruct((E2, O2), F32),
    )(e1p, d2, esum, nsum, u, gbW_e, gbW_n, gbW_u, gb_b, W1e2, W1g, dec_b1,
      W22, b22)


def _sc_edge(eap2, xsr, s3d, r3d):
    E2, H2 = eap2.shape
    E = 2 * E2
    H = H2 // 2
    N = xsr.shape[0]
    epw = E // NW
    nch = epw // C
    nrb = N // RB
    nzi = (nrb + NS - 1) // NS
    mesh = plsc.VectorSubcoreMesh(core_axis_name="c", subcore_axis_name="s")

    @functools.partial(
        pl.kernel,
        out_type=[
            jax.ShapeDtypeStruct((E2, H2), F32),
            jax.ShapeDtypeStruct((NC, N, H), F32),
        ],
        mesh=mesh,
        compiler_params=pltpu.CompilerParams(use_tc_tiling_on_sc=False),
        scratch_types=[
            pltpu.VMEM((nch, C), jnp.int32),
            pltpu.VMEM((nch, C), jnp.int32),
            pltpu.VMEM((C // 2, H2), F32),
            pltpu.VMEM((C, H2), F32),
            pltpu.VMEM((C, H2), F32),
            pltpu.VMEM((C // 2, H2), F32),
            pltpu.VMEM((C, H), F32),
            pltpu.VMEM((RB, H), F32),
            pltpu.VMEM_SHARED((N, H), F32),
            pltpu.SemaphoreType.DMA,
            pltpu.SemaphoreType.DMA,
        ],
    )
    def k(eap_hbm, xsr_hbm, s3d_hbm, r3d_hbm, e1_hbm, agg_hbm,
          idx_s, idx_r, bufE, bufS, bufR, bufP, bufU, zbuf, acc,
          sem1, sem2):
        cid = lax.axis_index("c")
        sid = lax.axis_index("s")
        wid = sid * NC + cid

        def zrow(i, carry):
            r = i // (H // 16)
            cc = i % (H // 16)
            zbuf[r, pl.ds(cc * 16, 16)] = jnp.zeros((16,), F32)
            return carry

        lax.fori_loop(0, RB * (H // 16), zrow, 0)

        def zacc(i, carry):
            b = sid * nzi + i

            @pl.when(b < nrb)
            def _():
                r0 = pl.multiple_of(b * RB, 8)
                pltpu.sync_copy(zbuf, acc.at[pl.ds(r0, RB)])

            return carry

        lax.fori_loop(0, nzi, zacc, 0)

        pltpu.sync_copy(s3d_hbm.at[wid], idx_s)
        pltpu.sync_copy(r3d_hbm.at[wid], idx_r)
        plsc.subcore_barrier()
        base_e = wid * epw

        def chunk(j, carry):
            e0h = pl.multiple_of((base_e + j * C) // 2, 8)
            pltpu.sync_copy(eap_hbm.at[pl.ds(e0h, C // 2)], bufE)
            cpS = pltpu.async_copy(xsr_hbm.at[idx_s.at[j]], bufS, sem1)
            cpR = pltpu.async_copy(xsr_hbm.at[idx_r.at[j]], bufR, sem2)
            cpS.wait()
            cpR.wait()

            def row(r2, rc):
                for half in range(2):
                    for cc in range(H // 16):
                        pk = pl.ds(half * H + cc * 16, 16)
                        sl = pl.ds(cc * 16, 16)
                        sh = pl.ds(H + cc * 16, 16)
                        e = 2 * r2 + half
                        v = bufE[r2, pk] + bufS[e, sl] + bufR[e, sh]
                        v = jnp.maximum(v, 0.0)
                        bufU[e, sl] = v
                        bufP[r2, pk] = v
                return rc

            lax.fori_loop(0, C // 2, row, 0)
            pltpu.sync_copy(bufP, e1_hbm.at[pl.ds(e0h, C // 2)])
            pltpu.sync_copy(bufU, acc.at[idx_r.at[j]], add=True)
            return carry

        lax.fori_loop(0, nch, chunk, 0)
        plsc.subcore_barrier()

        def flush(i, carry):
            b = sid * nzi + i

            @pl.when(b < nrb)
            def _():
                r0 = pl.multiple_of(b * RB, 8)
                pltpu.sync_copy(acc.at[pl.ds(r0, RB)], zbuf)
                pltpu.sync_copy(zbuf, agg_hbm.at[cid, pl.ds(r0, RB)])

            return carry

        lax.fori_loop(0, nzi, flush, 0)

    return k(eap2, xsr, s3d, r3d)


def _sc_dec_gather(nsnr, s3d, r3d, E):
    N, H2 = nsnr.shape
    H = H2 // 2
    epw = E // NW
    nch = epw // C
    mesh = plsc.VectorSubcoreMesh(core_axis_name="c", subcore_axis_name="s")

    @functools.partial(
        pl.kernel,
        out_type=jax.ShapeDtypeStruct((E // 2, H2), F32),
        mesh=mesh,
        compiler_params=pltpu.CompilerParams(use_tc_tiling_on_sc=True),
        scratch_types=[
            pltpu.VMEM((nch, C), jnp.int32),
            pltpu.VMEM((nch, C), jnp.int32),
            pltpu.VMEM((C, H2), F32),
            pltpu.VMEM((C, H2), F32),
            pltpu.VMEM((C // 2, H2), F32),
            pltpu.SemaphoreType.DMA,
            pltpu.SemaphoreType.DMA,
        ],
    )
    def k(nsnr_hbm, s3d_hbm, r3d_hbm, d_hbm,
          idx_s, idx_r, bufA, bufB, bufP, sem1, sem2):
        cid = lax.axis_index("c")
        sid = lax.axis_index("s")
        wid = sid * NC + cid
        pltpu.sync_copy(s3d_hbm.at[wid], idx_s)
        pltpu.sync_copy(r3d_hbm.at[wid], idx_r)
        base_e = wid * epw

        def chunk(j, carry):
            cpA = pltpu.async_copy(nsnr_hbm.at[idx_s.at[j]], bufA, sem1)
            cpB = pltpu.async_copy(nsnr_hbm.at[idx_r.at[j]], bufB, sem2)
            cpA.wait()
            cpB.wait()

            def row(r2, rc):
                for half in range(2):
                    for cc in range(H // 16):
                        pk = pl.ds(half * H + cc * 16, 16)
                        sl = pl.ds(cc * 16, 16)
                        sh = pl.ds(H + cc * 16, 16)
                        e = 2 * r2 + half
                        bufP[r2, pk] = bufA[e, sl] + bufB[e, sh]
                return rc

            lax.fori_loop(0, C // 2, row, 0)
            e0h = pl.multiple_of((base_e + j * C) // 2, 8)
            pltpu.sync_copy(bufP, d_hbm.at[pl.ds(e0h, C // 2)])
            return carry

        lax.fori_loop(0, nch, chunk, 0)

    return k(nsnr, s3d, r3d)


def kernel(x, edge_index, edge_attr, u, eb_W, eb_b, nb_W, nb_b, gb_W, gb_b,
           dec_W1, dec_b1, dec_W2, dec_b2):
    N, D = x.shape
    E = edge_index.shape[1]
    De = edge_attr.shape[1]
    H = eb_W.shape[1]
    OUT = dec_W2.shape[1]
    assert E % (NW * C) == 0 and N % RB == 0

    senders = edge_index[0]
    receivers = edge_index[1]
    s3d = senders.reshape(NW, E // (NW * C), C)
    r3d = receivers.reshape(NW, E // (NW * C), C)

    Wea = eb_W[:De]
    Ws = eb_W[De:De + D]
    Wr = eb_W[De + D:De + 2 * D]
    Wu = eb_W[De + 2 * D:]
    Wagg = nb_W[:H]
    Wx = nb_W[H:H + D]
    Wnu = nb_W[H + D:]
    gbW_e = gb_W[:H]
    gbW_n = gb_W[H:2 * H]
    gbW_u = gb_W[2 * H:]
    W1e = dec_W1[:H]
    W1s = dec_W1[H:2 * H]
    W1r = dec_W1[2 * H:3 * H]
    W1g = dec_W1[3 * H:]

    zDe = jnp.zeros((De, H), F32)
    Wea2 = jnp.concatenate([
        jnp.concatenate([Wea, zDe], axis=1),
        jnp.concatenate([zDe, Wea], axis=1)], axis=0)
    zH = jnp.zeros((H, H), F32)
    W1e2 = jnp.concatenate([
        jnp.concatenate([W1e, zH], axis=1),
        jnp.concatenate([zH, W1e], axis=1)], axis=0)
    zO = jnp.zeros((H, OUT), F32)
    W22 = jnp.concatenate([
        jnp.concatenate([dec_W2, zO], axis=1),
        jnp.concatenate([zO, dec_W2], axis=1)], axis=0)
    b22 = jnp.concatenate([dec_b2, dec_b2]).reshape(1, 2 * OUT)

    u2 = u.reshape(1, H)
    eb_b2 = eb_b.reshape(1, H)
    nb_b2 = nb_b.reshape(1, H)
    gb_b2 = gb_b.reshape(1, H)
    dec_b12 = dec_b1.reshape(1, H)

    ea2 = edge_attr.reshape(E // 2, 2 * De)

    xsr = _node_proj(x, Ws, Wr)
    eap2 = _edge_proj(ea2, Wea2, u2, Wu, eb_b2)
    e1p, aggP = _sc_edge(eap2, xsr, s3d, r3d)
    nsnr, nsum, esum = _node_block(aggP, x, Wagg, Wx, u2, Wnu, nb_b2,
                                   W1s, W1r)
    d2 = _sc_dec_gather(nsnr, s3d, r3d, E)
    out2 = _decoder(e1p, d2, esum, nsum, u2, gbW_e, gbW_n, gbW_u, gb_b2,
                    W1e2, W1g, dec_b12, W22, b22, E, N)
    return out2.reshape(E, OUT)


# speedup vs baseline: 1.9021x; 1.9021x over previous
import functools

import jax
import jax.numpy as jnp
from jax import lax
from jax.experimental import pallas as pl
from jax.experimental.pallas import tpu as pltpu
from jax.experimental.pallas import tpu_sc as plsc

F32 = jnp.float32

NC = 2
NS = 16
NW = NC * NS
C = 80
RB = 40


def _node_proj(x, Ws, Wr, bn=1000):
    N, D = x.shape
    H = Ws.shape[1]

    def body(x_ref, ws_ref, wr_ref, xs_ref, xr_ref):
        xb = x_ref[...]
        xs_ref[...] = jnp.dot(xb, ws_ref[...], preferred_element_type=F32)
        xr_ref[...] = jnp.dot(xb, wr_ref[...], preferred_element_type=F32)

    return pl.pallas_call(
        body,
        grid=(N // bn,),
        in_specs=[
            pl.BlockSpec((bn, D), lambda i: (i, 0)),
            pl.BlockSpec((D, H), lambda i: (0, 0)),
            pl.BlockSpec((D, H), lambda i: (0, 0)),
        ],
        out_specs=[
            pl.BlockSpec((bn, H), lambda i: (i, 0)),
            pl.BlockSpec((bn, H), lambda i: (i, 0)),
        ],
        out_shape=[
            jax.ShapeDtypeStruct((N, H), F32),
            jax.ShapeDtypeStruct((N, H), F32),
        ],
    )(x, Ws, Wr)


def _edge_proj(ea2, Wea2, u, Wu, eb_b, be=2000):
    E2, De2 = ea2.shape
    H = Wu.shape[1]

    def body(ea_ref, wea_ref, u_ref, wu_ref, b_ref, out_ref):
        c1 = jnp.dot(u_ref[...], wu_ref[...], preferred_element_type=F32) + b_ref[...]
        c2 = jnp.concatenate([c1, c1], axis=1)
        out_ref[...] = (
            jnp.dot(ea_ref[...], wea_ref[...], preferred_element_type=F32) + c2
        )

    return pl.pallas_call(
        body,
        grid=(E2 // be,),
        in_specs=[
            pl.BlockSpec((be, De2), lambda i: (i, 0)),
            pl.BlockSpec((De2, 2 * H), lambda i: (0, 0)),
            pl.BlockSpec((1, H), lambda i: (0, 0)),
            pl.BlockSpec((H, H), lambda i: (0, 0)),
            pl.BlockSpec((1, H), lambda i: (0, 0)),
        ],
        out_specs=pl.BlockSpec((be, 2 * H), lambda i: (i, 0)),
        out_shape=jax.ShapeDtypeStruct((E2, 2 * H), F32),
    )(ea2, Wea2, u, Wu, eb_b)


def _node_block(aggP, x, Wagg, Wx, u, Wnu, nb_b, W1s, W1r, bn=1000):
    N, D = x.shape
    H = Wagg.shape[1]

    def body(aggP_ref, x_ref, wagg_ref, wx_ref, u_ref, wnu_ref, b_ref,
             w1s_ref, w1r_ref, ns_ref, nr_ref, nsum_ref, esum_ref):
        agg = aggP_ref[0] + aggP_ref[1]
        cn = jnp.dot(u_ref[...], wnu_ref[...], preferred_element_type=F32) + b_ref[...]
        n1 = jnp.maximum(
            jnp.dot(agg, wagg_ref[...], preferred_element_type=F32)
            + jnp.dot(x_ref[...], wx_ref[...], preferred_element_type=F32)
            + cn, 0.0)
        ns_ref[...] = jnp.dot(n1, w1s_ref[...], preferred_element_type=F32)
        nr_ref[...] = jnp.dot(n1, w1r_ref[...], preferred_element_type=F32)

        @pl.when(pl.program_id(0) == 0)
        def _():
            nsum_ref[...] = jnp.zeros_like(nsum_ref)
            esum_ref[...] = jnp.zeros_like(esum_ref)

        nsum_ref[...] += jnp.sum(n1, axis=0, keepdims=True)
        esum_ref[...] += jnp.sum(agg, axis=0, keepdims=True)

    return pl.pallas_call(
        body,
        grid=(N // bn,),
        in_specs=[
            pl.BlockSpec((NC, bn, H), lambda i: (0, i, 0)),
            pl.BlockSpec((bn, D), lambda i: (i, 0)),
            pl.BlockSpec((H, H), lambda i: (0, 0)),
            pl.BlockSpec((D, H), lambda i: (0, 0)),
            pl.BlockSpec((1, H), lambda i: (0, 0)),
            pl.BlockSpec((H, H), lambda i: (0, 0)),
            pl.BlockSpec((1, H), lambda i: (0, 0)),
            pl.BlockSpec((H, H), lambda i: (0, 0)),
            pl.BlockSpec((H, H), lambda i: (0, 0)),
        ],
        out_specs=[
            pl.BlockSpec((bn, H), lambda i: (i, 0)),
            pl.BlockSpec((bn, H), lambda i: (i, 0)),
            pl.BlockSpec((1, H), lambda i: (0, 0)),
            pl.BlockSpec((1, H), lambda i: (0, 0)),
        ],
        out_shape=[
            jax.ShapeDtypeStruct((N, H), F32),
            jax.ShapeDtypeStruct((N, H), F32),
            jax.ShapeDtypeStruct((1, H), F32),
            jax.ShapeDtypeStruct((1, H), F32),
        ],
    )(aggP, x, Wagg, Wx, u, Wnu, nb_b, W1s, W1r)


def _decoder(e1p, d2, esum, nsum, u, gbW_e, gbW_n, gbW_u, gb_b, W1e2, W1g,
             dec_b1, W22, b22, n_edges, n_nodes, be=2000):
    E2, H2 = e1p.shape
    H = H2 // 2
    O2 = W22.shape[1]

    def body(e1_ref, d_ref, esum_ref, nsum_ref, u_ref, gbe_ref, gbn_ref,
             gbu_ref, gbb_ref, w1e_ref, w1g_ref, b1_ref, w2_ref, b2_ref,
             out_ref):
        g1 = jnp.maximum(
            jnp.dot(esum_ref[...] * (1.0 / n_edges), gbe_ref[...],
                    preferred_element_type=F32)
            + jnp.dot(nsum_ref[...] * (1.0 / n_nodes), gbn_ref[...],
                      preferred_element_type=F32)
            + jnp.dot(u_ref[...], gbu_ref[...], preferred_element_type=F32)
            + gbb_ref[...], 0.0)
        cdec = jnp.dot(g1, w1g_ref[...], preferred_element_type=F32) + b1_ref[...]
        cdec2 = jnp.concatenate([cdec, cdec], axis=1)
        h = jnp.maximum(
            jnp.dot(e1_ref[...], w1e_ref[...], preferred_element_type=F32)
            + d_ref[...] + cdec2, 0.0)
        out_ref[...] = jnp.dot(h, w2_ref[...], preferred_element_type=F32) + b2_ref[...]

    return pl.pallas_call(
        body,
        grid=(E2 // be,),
        in_specs=[
            pl.BlockSpec((be, H2), lambda i: (i, 0)),
            pl.BlockSpec((be, H2), lambda i: (i, 0)),
            pl.BlockSpec((1, H), lambda i: (0, 0)),
            pl.BlockSpec((1, H), lambda i: (0, 0)),
            pl.BlockSpec((1, H), lambda i: (0, 0)),
            pl.BlockSpec((H, H), lambda i: (0, 0)),
            pl.BlockSpec((H, H), lambda i: (0, 0)),
            pl.BlockSpec((H, H), lambda i: (0, 0)),
            pl.BlockSpec((1, H), lambda i: (0, 0)),
            pl.BlockSpec((H2, H2), lambda i: (0, 0)),
            pl.BlockSpec((H, H), lambda i: (0, 0)),
            pl.BlockSpec((1, H), lambda i: (0, 0)),
            pl.BlockSpec((H2, O2), lambda i: (0, 0)),
            pl.BlockSpec((1, O2), lambda i: (0, 0)),
        ],
        out_specs=pl.BlockSpec((be, O2), lambda i: (i, 0)),
        out_shape=jax.ShapeDtypeStruct((E2, O2), F32),
    )(e1p, d2, esum, nsum, u, gbW_e, gbW_n, gbW_u, gb_b, W1e2, W1g, dec_b1,
      W22, b22)


def _sc_edge(eap2, xs, xr, s3d, r3d):
    E2, H2 = eap2.shape
    E = 2 * E2
    H = H2 // 2
    N = xs.shape[0]
    epw = E // NW
    nch = epw // C
    nrb = N // RB
    nzi = (nrb + NS - 1) // NS
    mesh = plsc.VectorSubcoreMesh(core_axis_name="c", subcore_axis_name="s")

    @functools.partial(
        pl.kernel,
        out_type=[
            jax.ShapeDtypeStruct((E2, H2), F32),
            jax.ShapeDtypeStruct((NC, N, H), F32),
        ],
        mesh=mesh,
        compiler_params=pltpu.CompilerParams(use_tc_tiling_on_sc=False),
        scratch_types=[
            pltpu.VMEM((nch, C), jnp.int32),
            pltpu.VMEM((nch, C), jnp.int32),
            pltpu.VMEM((C // 2, H2), F32),
            pltpu.VMEM((C // 2, H2), F32),
            pltpu.VMEM((C, H), F32),
            pltpu.VMEM((C, H), F32),
            pltpu.VMEM((C, H), F32),
            pltpu.VMEM((C, H), F32),
            pltpu.VMEM((C // 2, H2), F32),
            pltpu.VMEM((C // 2, H2), F32),
            pltpu.VMEM((C, H), F32),
            pltpu.VMEM((RB, H), F32),
            pltpu.VMEM_SHARED((N, H), F32),
            pltpu.SemaphoreType.DMA,
            pltpu.SemaphoreType.DMA,
            pltpu.SemaphoreType.DMA,
            pltpu.SemaphoreType.DMA,
            pltpu.SemaphoreType.DMA,
            pltpu.SemaphoreType.DMA,
            pltpu.SemaphoreType.DMA,
            pltpu.SemaphoreType.DMA,
        ],
    )
    def k(eap_hbm, xs_hbm, xr_hbm, s3d_hbm, r3d_hbm, e1_hbm, agg_hbm,
          idx_s, idx_r, bufE0, bufE1, bufS0, bufS1, bufR0, bufR1,
          bufP0, bufP1, bufU, zbuf, acc,
          semE0, semE1, semS0, semS1, semR0, semR1, semW0, semW1):
        cid = lax.axis_index("c")
        sid = lax.axis_index("s")
        wid = sid * NC + cid

        def zrow(i, carry):
            r = i // (H // 16)
            cc = i % (H // 16)
            zbuf[r, pl.ds(cc * 16, 16)] = jnp.zeros((16,), F32)
            return carry

        lax.fori_loop(0, RB * (H // 16), zrow, 0)

        def zacc(i, carry):
            b = sid * nzi + i

            @pl.when(b < nrb)
            def _():
                r0 = pl.multiple_of(b * RB, 8)
                pltpu.sync_copy(zbuf, acc.at[pl.ds(r0, RB)])

            return carry

        lax.fori_loop(0, nzi, zacc, 0)

        pltpu.sync_copy(s3d_hbm.at[wid], idx_s)
        pltpu.sync_copy(r3d_hbm.at[wid], idx_r)
        plsc.subcore_barrier()
        base_e = wid * epw

        def start_in(j, bE, bS, bR, sE, sS, sR):
            e0h = pl.multiple_of((base_e + j * C) // 2, 8)
            pltpu.async_copy(eap_hbm.at[pl.ds(e0h, C // 2)], bE, sE)
            pltpu.async_copy(xs_hbm.at[idx_s.at[j]], bS, sS)
            pltpu.async_copy(xr_hbm.at[idx_r.at[j]], bR, sR)

        def wait_in(bE, bS, bR, sE, sS, sR):
            pltpu.make_async_copy(eap_hbm.at[pl.ds(0, C // 2)], bE, sE).wait()
            pltpu.make_async_copy(xs_hbm.at[idx_s.at[0]], bS, sS).wait()
            pltpu.make_async_copy(xr_hbm.at[idx_r.at[0]], bR, sR).wait()

        def compute(j, bE, bS, bR, bP):
            def row(r2, rc):
                for half in range(2):
                    for cc in range(H // 16):
                        pk = pl.ds(half * H + cc * 16, 16)
                        sl = pl.ds(cc * 16, 16)
                        e = 2 * r2 + half
                        v = bE[r2, pk] + bS[e, sl] + bR[e, sl]
                        v = jnp.maximum(v, 0.0)
                        bufU[e, sl] = v
                        bP[r2, pk] = v
                return rc

            lax.fori_loop(0, C // 2, row, 0)
            pltpu.sync_copy(bufU, acc.at[idx_r.at[j]], add=True)

        def start_out(j, bP, sW):
            e0h = pl.multiple_of((base_e + j * C) // 2, 8)
            pltpu.async_copy(bP, e1_hbm.at[pl.ds(e0h, C // 2)], sW)

        def wait_out(bP, sW):
            pltpu.make_async_copy(bP, e1_hbm.at[pl.ds(0, C // 2)], sW).wait()

        set0 = (bufE0, bufS0, bufR0, semE0, semS0, semR0)
        set1 = (bufE1, bufS1, bufR1, semE1, semS1, semR1)
        start_in(0, *set0)

        def pipe(jj, carry):
            j0 = 2 * jj

            @pl.when(j0 + 1 < nch)
            def _():
                start_in(j0 + 1, *set1)

            wait_in(*set0)

            @pl.when(jj > 0)
            def _():
                wait_out(bufP0, semW0)

            compute(j0, bufE0, bufS0, bufR0, bufP0)
            start_out(j0, bufP0, semW0)

            @pl.when(j0 + 2 < nch)
            def _():
                start_in(j0 + 2, *set0)

            @pl.when(j0 + 1 < nch)
            def _():
                wait_in(*set1)

                @pl.when(jj > 0)
                def _():
                    wait_out(bufP1, semW1)

                compute(j0 + 1, bufE1, bufS1, bufR1, bufP1)
                start_out(j0 + 1, bufP1, semW1)

            return carry

        lax.fori_loop(0, (nch + 1) // 2, pipe, 0)
        wait_out(bufP0, semW0)
        wait_out(bufP1, semW1)
        plsc.subcore_barrier()

        def flush(i, carry):
            b = sid * nzi + i

            @pl.when(b < nrb)
            def _():
                r0 = pl.multiple_of(b * RB, 8)
                pltpu.sync_copy(acc.at[pl.ds(r0, RB)], zbuf)
                pltpu.sync_copy(zbuf, agg_hbm.at[cid, pl.ds(r0, RB)])

            return carry

        lax.fori_loop(0, nzi, flush, 0)

    return k(eap2, xs, xr, s3d, r3d)


def _sc_dec_gather(ns, nr, s3d, r3d, E):
    N, H = ns.shape
    H2 = 2 * H
    epw = E // NW
    nch = epw // C
    mesh = plsc.VectorSubcoreMesh(core_axis_name="c", subcore_axis_name="s")

    @functools.partial(
        pl.kernel,
        out_type=jax.ShapeDtypeStruct((E // 2, H2), F32),
        mesh=mesh,
        compiler_params=pltpu.CompilerParams(use_tc_tiling_on_sc=False),
        scratch_types=[
            pltpu.VMEM((nch, C), jnp.int32),
            pltpu.VMEM((nch, C), jnp.int32),
            pltpu.VMEM((C, H), F32),
            pltpu.VMEM((C, H), F32),
            pltpu.VMEM((C, H), F32),
            pltpu.VMEM((C, H), F32),
            pltpu.VMEM((C // 2, H2), F32),
            pltpu.VMEM((C // 2, H2), F32),
            pltpu.SemaphoreType.DMA,
            pltpu.SemaphoreType.DMA,
            pltpu.SemaphoreType.DMA,
            pltpu.SemaphoreType.DMA,
            pltpu.SemaphoreType.DMA,
            pltpu.SemaphoreType.DMA,
        ],
    )
    def k(ns_hbm, nr_hbm, s3d_hbm, r3d_hbm, d_hbm,
          idx_s, idx_r, bufA0, bufA1, bufB0, bufB1, bufP0, bufP1,
          semA0, semA1, semB0, semB1, semW0, semW1):
        cid = lax.axis_index("c")
        sid = lax.axis_index("s")
        wid = sid * NC + cid
        pltpu.sync_copy(s3d_hbm.at[wid], idx_s)
        pltpu.sync_copy(r3d_hbm.at[wid], idx_r)
        base_e = wid * epw

        def start_in(j, bA, bB, sA, sB):
            pltpu.async_copy(ns_hbm.at[idx_s.at[j]], bA, sA)
            pltpu.async_copy(nr_hbm.at[idx_r.at[j]], bB, sB)

        def wait_in(bA, bB, sA, sB):
            pltpu.make_async_copy(ns_hbm.at[idx_s.at[0]], bA, sA).wait()
            pltpu.make_async_copy(nr_hbm.at[idx_r.at[0]], bB, sB).wait()

        def compute(bA, bB, bP):
            def row(r2, rc):
                for half in range(2):
                    for cc in range(H // 16):
                        pk = pl.ds(half * H + cc * 16, 16)
                        sl = pl.ds(cc * 16, 16)
                        e = 2 * r2 + half
                        bP[r2, pk] = bA[e, sl] + bB[e, sl]
                return rc

            lax.fori_loop(0, C // 2, row, 0)

        def start_out(j, bP, sW):
            e0h = pl.multiple_of((base_e + j * C) // 2, 8)
            pltpu.async_copy(bP, d_hbm.at[pl.ds(e0h, C // 2)], sW)

        def wait_out(bP, sW):
            pltpu.make_async_copy(bP, d_hbm.at[pl.ds(0, C // 2)], sW).wait()

        set0 = (bufA0, bufB0, semA0, semB0)
        set1 = (bufA1, bufB1, semA1, semB1)
        start_in(0, *set0)

        def pipe(jj, carry):
            j0 = 2 * jj

            @pl.when(j0 + 1 < nch)
            def _():
                start_in(j0 + 1, *set1)

            wait_in(*set0)

            @pl.when(jj > 0)
            def _():
                wait_out(bufP0, semW0)

            compute(bufA0, bufB0, bufP0)
            start_out(j0, bufP0, semW0)

            @pl.when(j0 + 2 < nch)
            def _():
                start_in(j0 + 2, *set0)

            @pl.when(j0 + 1 < nch)
            def _():
                wait_in(*set1)

                @pl.when(jj > 0)
                def _():
                    wait_out(bufP1, semW1)

                compute(bufA1, bufB1, bufP1)
                start_out(j0 + 1, bufP1, semW1)

            return carry

        lax.fori_loop(0, (nch + 1) // 2, pipe, 0)
        wait_out(bufP0, semW0)
        wait_out(bufP1, semW1)

    return k(ns, nr, s3d, r3d)


def kernel(x, edge_index, edge_attr, u, eb_W, eb_b, nb_W, nb_b, gb_W, gb_b,
           dec_W1, dec_b1, dec_W2, dec_b2):
    N, D = x.shape
    E = edge_index.shape[1]
    De = edge_attr.shape[1]
    H = eb_W.shape[1]
    OUT = dec_W2.shape[1]
    assert E % (NW * C) == 0 and N % RB == 0

    senders = edge_index[0]
    receivers = edge_index[1]
    s3d = senders.reshape(NW, E // (NW * C), C)
    r3d = receivers.reshape(NW, E // (NW * C), C)

    Wea = eb_W[:De]
    Ws = eb_W[De:De + D]
    Wr = eb_W[De + D:De + 2 * D]
    Wu = eb_W[De + 2 * D:]
    Wagg = nb_W[:H]
    Wx = nb_W[H:H + D]
    Wnu = nb_W[H + D:]
    gbW_e = gb_W[:H]
    gbW_n = gb_W[H:2 * H]
    gbW_u = gb_W[2 * H:]
    W1e = dec_W1[:H]
    W1s = dec_W1[H:2 * H]
    W1r = dec_W1[2 * H:3 * H]
    W1g = dec_W1[3 * H:]

    zDe = jnp.zeros((De, H), F32)
    Wea2 = jnp.concatenate([
        jnp.concatenate([Wea, zDe], axis=1),
        jnp.concatenate([zDe, Wea], axis=1)], axis=0)
    zH = jnp.zeros((H, H), F32)
    W1e2 = jnp.concatenate([
        jnp.concatenate([W1e, zH], axis=1),
        jnp.concatenate([zH, W1e], axis=1)], axis=0)
    zO = jnp.zeros((H, OUT), F32)
    W22 = jnp.concatenate([
        jnp.concatenate([dec_W2, zO], axis=1),
        jnp.concatenate([zO, dec_W2], axis=1)], axis=0)
    b22 = jnp.concatenate([dec_b2, dec_b2]).reshape(1, 2 * OUT)

    u2 = u.reshape(1, H)
    eb_b2 = eb_b.reshape(1, H)
    nb_b2 = nb_b.reshape(1, H)
    gb_b2 = gb_b.reshape(1, H)
    dec_b12 = dec_b1.reshape(1, H)

    ea2 = edge_attr.reshape(E // 2, 2 * De)

    xs, xr = _node_proj(x, Ws, Wr)
    eap2 = _edge_proj(ea2, Wea2, u2, Wu, eb_b2)
    e1p, aggP = _sc_edge(eap2, xs, xr, s3d, r3d)
    ns, nr, nsum, esum = _node_block(aggP, x, Wagg, Wx, u2, Wnu, nb_b2,
                                     W1s, W1r)
    d2 = _sc_dec_gather(ns, nr, s3d, r3d, E)
    out2 = _decoder(e1p, d2, esum, nsum, u2, gbW_e, gbW_n, gbW_u, gb_b2,
                    W1e2, W1g, dec_b12, W22, b22, E, N)
    return out2.reshape(E, OUT)
